# baseline (speedup 1.0000x reference)
import math

import jax
import jax.numpy as jnp
from jax.experimental import pallas as pl

_SCALE = 32.0
_MARGIN = 0.2
_COS_M = math.cos(_MARGIN)
_SIN_M = math.sin(_MARGIN)
_TH = math.cos(math.pi - _MARGIN)
_MMM = 1.0 + math.cos(math.pi - _MARGIN)

_RB = 1024
_CB = 2048


def _body(cos_ref, lab_ref, out_ref):
    j = pl.program_id(1)
    cos = cos_ref[...]
    lab = lab_ref[...]
    col = jax.lax.broadcasted_iota(jnp.int32, cos.shape, 1) + j * _CB
    mask = col == lab
    g = jnp.sum(jnp.where(mask, cos, 0.0), axis=1, keepdims=True)
    sine = jnp.sqrt(1.0 - g * g)
    ph = g * _COS_M - sine * _SIN_M
    ph = jnp.where(g > _TH, ph, g - _MMM)
    out_ref[...] = jnp.where(mask, ph, cos) * _SCALE


def kernel(cosine, label):
    B, C = cosine.shape
    lab2 = label.astype(jnp.int32).reshape(B, 1)
    grid = (B // _RB, pl.cdiv(C, _CB))
    return pl.pallas_call(
        _body,
        grid=grid,
        in_specs=[
            pl.BlockSpec((_RB, _CB), lambda i, j: (i, j)),
            pl.BlockSpec((_RB, 1), lambda i, j: (i, 0)),
        ],
        out_specs=pl.BlockSpec((_RB, _CB), lambda i, j: (i, j)),
        out_shape=jax.ShapeDtypeStruct((B, C), jnp.float32),
    )(cosine, lab2)


# speedup vs baseline: 1.0021x; 1.0021x over previous
import math

import jax
import jax.numpy as jnp
from jax.experimental import pallas as pl

_SCALE = 32.0
_MARGIN = 0.2
_COS_M = math.cos(_MARGIN)
_SIN_M = math.sin(_MARGIN)
_TH = math.cos(math.pi - _MARGIN)
_MMM = 1.0 + math.cos(math.pi - _MARGIN)

_RB = 512
_CB = 4096


def _body(cos_ref, lab_ref, out_ref):
    j = pl.program_id(1)
    cos = cos_ref[...]
    lab = lab_ref[...]
    col = jax.lax.broadcasted_iota(jnp.int32, cos.shape, 1) + j * _CB
    mask = col == lab
    g = jnp.sum(jnp.where(mask, cos, 0.0), axis=1, keepdims=True)
    sine = jnp.sqrt(1.0 - g * g)
    ph = g * _COS_M - sine * _SIN_M
    ph = jnp.where(g > _TH, ph, g - _MMM)
    out_ref[...] = jnp.where(mask, ph, cos) * _SCALE


def kernel(cosine, label):
    B, C = cosine.shape
    lab2 = label.astype(jnp.int32).reshape(B, 1)
    grid = (B // _RB, pl.cdiv(C, _CB))
    return pl.pallas_call(
        _body,
        grid=grid,
        in_specs=[
            pl.BlockSpec((_RB, _CB), lambda i, j: (i, j)),
            pl.BlockSpec((_RB, 1), lambda i, j: (i, 0)),
        ],
        out_specs=pl.BlockSpec((_RB, _CB), lambda i, j: (i, j)),
        out_shape=jax.ShapeDtypeStruct((B, C), jnp.float32),
    )(cosine, lab2)
